# baseline (speedup 1.0000x reference)
import jax
import jax.numpy as jnp
from jax.experimental import pallas as pl

MAXW = 10


def _greedy(order, starts, ends, valid, num_top, num_tokens):
    num_cands = order.shape[0]
    offsets = jnp.arange(MAXW, dtype=jnp.int32)

    def body(i, state):
        sel, count, s2e, e2s = state
        cid = order[i]
        s = starts[cid]
        e = ends[cid]
        ok = valid[cid] & (count < num_top)
        ts = s + offsets
        tmask = ts <= e
        tsc = jnp.minimum(ts, num_tokens - 1)
        me = s2e[tsc]
        ms = e2s[tsc]
        cross1 = jnp.any(tmask & (ts > s) & (me > e))
        cross2 = jnp.any(tmask & (ts < e) & (ms >= 0) & (ms < s))
        take = ok & jnp.logical_not(cross1 | cross2)
        sel = jnp.where(take, sel.at[count].set(cid), sel)
        new_me = jnp.maximum(s2e[s], e)
        s2e = jnp.where(take, s2e.at[s].set(new_me), s2e)
        old_ms = e2s[e]
        new_ms = jnp.where((old_ms == -1) | (s < old_ms), s, old_ms)
        e2s = jnp.where(take, e2s.at[e].set(new_ms), e2s)
        count = count + jnp.where(take, jnp.int32(1), jnp.int32(0))
        return sel, count, s2e, e2s

    sel0 = jnp.full((num_top,), -1, dtype=jnp.int32)
    s2e0 = jnp.full((num_tokens,), -1, dtype=jnp.int32)
    e2s0 = jnp.full((num_tokens,), -1, dtype=jnp.int32)
    sel, count, _, _ = jax.lax.fori_loop(0, num_cands, body, (sel0, jnp.int32(0), s2e0, e2s0))
    slot = jnp.arange(num_top, dtype=jnp.int32)
    filled = slot < count
    key = jnp.where(filled,
                    jnp.take(starts, sel, mode='clip') * jnp.int32(num_tokens + MAXW)
                    + jnp.take(ends, sel, mode='clip'),
                    jnp.int32(jnp.iinfo(jnp.int32).max))
    perm = jnp.argsort(key)
    sel_sorted = jnp.take(sel, perm)
    sel_sorted = jnp.where(jnp.take(filled, perm), sel_sorted, sel_sorted[0])
    return sel_sorted


def _emdot(a, b):
    return jnp.dot(a.astype(jnp.bfloat16), b.astype(jnp.bfloat16),
                   preferred_element_type=jnp.float32)


def kernel(token_emb, sentence_map, attn_w, attn_b, width_emb, W1, b1, W2, b2):
    N = token_emb.shape[0]
    t = jnp.arange(N, dtype=jnp.int32)
    dd = jnp.arange(MAXW, dtype=jnp.int32)
    starts = jnp.repeat(t, MAXW)
    ends = starts + jnp.tile(dd, N)
    start_sent = jnp.take(sentence_map, starts, axis=0)
    end_sent = jnp.take(sentence_map, jnp.minimum(ends, N - 1), axis=0)
    valid = (ends < N) & (start_sent == end_sent)
    ce_safe = jnp.minimum(ends, N - 1)

    start_emb = jnp.take(token_emb, starts, axis=0)
    end_emb = jnp.take(token_emb, ce_safe, axis=0)
    w_emb = jnp.take(width_emb, ce_safe - starts, axis=0)
    token_attn = _emdot(token_emb, attn_w) + attn_b
    doc_range = jnp.arange(N)[None, :]
    span_mask = (doc_range >= starts[:, None]) & (doc_range <= ce_safe[:, None])
    attn_logits = jnp.where(span_mask, token_attn[None, :], jnp.float32(-1e30))
    attn_probs = jax.nn.softmax(attn_logits, axis=1)
    attended = _emdot(attn_probs, token_emb)
    span_emb = jnp.concatenate([start_emb, end_emb, w_emb, attended], axis=1)
    h = jax.nn.relu(_emdot(span_emb, W1) + b1)
    scores = (_emdot(h, W2) + b2)[:, 0]

    num_top = int(0.4 * N)
    sort_key = jnp.where(valid, scores, jnp.float32(-jnp.inf))
    order = jnp.argsort(-sort_key)
    sel_j = _greedy(order, starts, ends, valid, num_top, N)
    return (jnp.take(starts, sel_j), jnp.take(ends, sel_j), jnp.take(scores, sel_j))


def _impl(token_emb, sentence_map, attn_w, attn_b, width_emb, W1, b1, W2, b2):
    N, H = token_emb.shape
    F = W1.shape[1]
    W1s, W1e = W1[:H], W1[H:2 * H]
    W1w = W1[2 * H:2 * H + width_emb.shape[1]]
    W1a = W1[2 * H + width_emb.shape[1]:]
    Pstart = token_emb @ W1s
    Pend = token_emb @ W1e
    Patt = token_emb @ W1a
    Wtab = width_emb @ W1w
    ta = token_emb @ attn_w + attn_b

    t = jnp.arange(N, dtype=jnp.int32)
    dd = jnp.arange(MAXW, dtype=jnp.int32)
    tok = t[:, None] + dd[None, :]
    tok_c = jnp.minimum(tok, N - 1)
    m3 = (dd[None, None, :] <= dd[None, :, None]) & (tok[:, None, :] <= N - 1)
    a_win = ta[tok_c]
    logits = jnp.where(m3, a_win[:, None, :], -jnp.inf)
    mx = jnp.max(logits, axis=-1, keepdims=True)
    p = jnp.where(m3, jnp.exp(logits - mx), 0.0)
    Z = jnp.sum(p, axis=-1, keepdims=True)
    q = p / Z
    Patt_win = Patt[tok_c]
    A = jnp.einsum('twd,tdf->twf', q, Patt_win)
    hpre = Pstart[:, None, :] + Pend[tok_c] + Wtab[None, :, :] + A + b1
    scores = (jnp.maximum(hpre, 0.0).reshape(N * MAXW, F) @ W2 + b2)[:, 0]

    starts = jnp.repeat(t, MAXW)
    ends = starts + jnp.tile(dd, N)
    start_sent = jnp.take(sentence_map, starts, axis=0)
    end_sent = jnp.take(sentence_map, jnp.minimum(ends, N - 1), axis=0)
    valid = (ends < N) & (start_sent == end_sent)

    num_top = int(0.4 * N)
    sort_key = jnp.where(valid, scores, jnp.float32(-jnp.inf))
    order = jnp.argsort(-sort_key)
    sel_j = _greedy(order, starts, ends, valid, num_top, N)
    return (jnp.take(starts, sel_j), jnp.take(ends, sel_j), jnp.take(scores, sel_j))


# speedup vs baseline: 72.2613x; 72.2613x over previous
import functools

import jax
import jax.numpy as jnp
from jax import lax
from jax.experimental import pallas as pl
from jax.experimental.pallas import tpu as pltpu
from jax.experimental.pallas import tpu_sc as plsc

MAXW = 10
NT = 2048
NUM_TOP = 819
SEL_PAD = 832


def _greedy_sc(order_hbm, vcnt_hbm, sel_hbm, cnt_hbm, order_v, vcnt_v,
               sel_v, cnt_v, s2e_v, e2s_v):
    cid = lax.axis_index("c")
    sid = lax.axis_index("s")
    is_main = (cid == 0) & (sid == 0)

    if True:
        pltpu.sync_copy(order_hbm, order_v.at[pl.ds(0, order_hbm.shape[0])])
        pltpu.sync_copy(vcnt_hbm, vcnt_v)
        neg1 = jnp.full((16,), -1, jnp.int32)

        def init_maps(j, carry):
            s2e_v[pl.ds(j * 16, 16)] = neg1
            e2s_v[pl.ds(j * 16, 16)] = neg1
            return carry

        lax.fori_loop(0, NT // 16, init_maps, 0)

        def init_sel(j, carry):
            sel_v[pl.ds(j * 16, 16)] = neg1
            return carry

        lax.fori_loop(0, SEL_PAD // 16, init_sel, 0)

        v_lim = vcnt_v[...][0]
        iota = lax.iota(jnp.int32, 16)
        lane0 = iota < 1

        def body(i, count):
            pack = order_v[pl.ds(i, 16)][0]
            s = pack >> 12
            e = pack & 4095
            ts = s + iota
            tmask = ts <= e
            tsc = jnp.minimum(ts, NT - 1)
            me = plsc.load_gather(s2e_v, [tsc])
            ms = plsc.load_gather(e2s_v, [tsc])
            bad = ((tmask & (ts > s) & (me > e))
                   | (tmask & (ts < e) & (ms >= 0) & (ms < s)))
            nbad = plsc.all_reduce_population_count(bad)[0]
            take = (i < v_lim) & (count < NUM_TOP) & (nbad == 0)
            wmask = lane0 & take

            sv = jnp.full((16,), s, jnp.int32)
            ev = jnp.full((16,), jnp.minimum(e, NT - 1), jnp.int32)
            plsc.store_scatter(sel_v, [jnp.full((16,), count, jnp.int32)],
                               jnp.full((16,), 9 * s + e, jnp.int32),
                               mask=wmask)
            new_me = jnp.maximum(me[0], e)
            plsc.store_scatter(s2e_v, [sv],
                               jnp.full((16,), new_me, jnp.int32),
                               mask=wmask)
            old_ms = plsc.load_gather(e2s_v, [ev])[0]
            new_ms = jnp.where((old_ms == -1) | (s < old_ms), s, old_ms)
            plsc.store_scatter(e2s_v, [ev],
                               jnp.full((16,), new_ms, jnp.int32),
                               mask=wmask)

            return count + jnp.where(take, 1, 0).astype(jnp.int32)

        count = lax.fori_loop(0, order_hbm.shape[0], body, jnp.int32(0))
        cnt_v[pl.ds(0, 16)] = jnp.full((16,), count, jnp.int32)

        @pl.when(is_main)
        def _():
            pltpu.sync_copy(sel_v, sel_hbm)
            pltpu.sync_copy(cnt_v, cnt_hbm)


@jax.jit
def _greedy_call(order_packed, vcnt):
    mesh = plsc.VectorSubcoreMesh(core_axis_name="c", subcore_axis_name="s")
    return pl.kernel(
        _greedy_sc,
        out_type=(jax.ShapeDtypeStruct((SEL_PAD,), jnp.int32),
                  jax.ShapeDtypeStruct((16,), jnp.int32)),
        mesh=mesh,
        compiler_params=pltpu.CompilerParams(needs_layout_passes=False),
        scratch_types=[
            pltpu.VMEM((order_packed.shape[0] + 16,), jnp.int32),
            pltpu.VMEM((16,), jnp.int32),
            pltpu.VMEM((SEL_PAD,), jnp.int32),
            pltpu.VMEM((16,), jnp.int32),
            pltpu.VMEM((NT,), jnp.int32),
            pltpu.VMEM((NT,), jnp.int32),
        ],
    )(order_packed, vcnt)


def _emdot(a, b):
    return jnp.dot(a.astype(jnp.bfloat16), b.astype(jnp.bfloat16),
                   preferred_element_type=jnp.float32)


def kernel(token_emb, sentence_map, attn_w, attn_b, width_emb, W1, b1, W2, b2):
    N = token_emb.shape[0]
    t = jnp.arange(N, dtype=jnp.int32)
    dd = jnp.arange(MAXW, dtype=jnp.int32)
    starts = jnp.repeat(t, MAXW)
    ends = starts + jnp.tile(dd, N)
    start_sent = jnp.take(sentence_map, starts, axis=0)
    end_sent = jnp.take(sentence_map, jnp.minimum(ends, N - 1), axis=0)
    valid = (ends < N) & (start_sent == end_sent)
    ce_safe = jnp.minimum(ends, N - 1)

    start_emb = jnp.take(token_emb, starts, axis=0)
    end_emb = jnp.take(token_emb, ce_safe, axis=0)
    w_emb = jnp.take(width_emb, ce_safe - starts, axis=0)
    token_attn = _emdot(token_emb, attn_w) + attn_b
    doc_range = jnp.arange(N)[None, :]
    span_mask = (doc_range >= starts[:, None]) & (doc_range <= ce_safe[:, None])
    attn_logits = jnp.where(span_mask, token_attn[None, :], jnp.float32(-1e30))
    attn_probs = jax.nn.softmax(attn_logits, axis=1)
    attended = _emdot(attn_probs, token_emb)
    span_emb = jnp.concatenate([start_emb, end_emb, w_emb, attended], axis=1)
    h = jax.nn.relu(_emdot(span_emb, W1) + b1)
    scores = (_emdot(h, W2) + b2)[:, 0]

    num_top = int(0.4 * N)
    sort_key = jnp.where(valid, scores, jnp.float32(-jnp.inf))
    order = jnp.argsort(-sort_key)
    vcnt = jnp.full((16,), jnp.sum(valid.astype(jnp.int32)), jnp.int32)
    packed = (jnp.take(starts, order) << 12) | jnp.take(ends, order)
    sel_pad, cnt = _greedy_call(packed.astype(jnp.int32), vcnt)
    sel = sel_pad[:num_top]
    count = cnt[0]

    slot = jnp.arange(num_top, dtype=jnp.int32)
    filled = slot < count
    key = jnp.where(filled,
                    jnp.take(starts, sel, mode='clip') * jnp.int32(N + MAXW)
                    + jnp.take(ends, sel, mode='clip'),
                    jnp.int32(jnp.iinfo(jnp.int32).max))
    perm = jnp.argsort(key)
    sel_sorted = jnp.take(sel, perm)
    sel_sorted = jnp.where(jnp.take(filled, perm), sel_sorted, sel_sorted[0])
    return (jnp.take(starts, sel_sorted), jnp.take(ends, sel_sorted),
            jnp.take(scores, sel_sorted))


# speedup vs baseline: 126.0493x; 1.7444x over previous
import functools

import jax
import jax.numpy as jnp
from jax import lax
from jax.experimental import pallas as pl
from jax.experimental.pallas import tpu as pltpu
from jax.experimental.pallas import tpu_sc as plsc

MAXW = 10
NT = 2048
NC = NT * MAXW
NUM_TOP = 819
SEL_PAD = 832
STAGE1 = 1536


def _greedy_loop(order_v, sel_v, s2e_v, e2s_v, i_lo, i_hi, count0, v_lim):
    iota = lax.iota(jnp.int32, 16)
    lane0 = iota < 1

    def body(i, count):
        pack = order_v[pl.ds(i, 16)][0]
        s = pack >> 12
        e = pack & 4095
        ts = s + iota
        tmask = ts <= e
        tsc = jnp.minimum(ts, NT - 1)
        me = plsc.load_gather(s2e_v, [tsc])
        ms = plsc.load_gather(e2s_v, [tsc])
        bad = ((tmask & (ts > s) & (me > e))
               | (tmask & (ts < e) & (ms >= 0) & (ms < s)))
        nbad = plsc.all_reduce_population_count(bad)[0]
        take = (count < NUM_TOP) & (nbad == 0)
        if v_lim is not None:
            take = take & (i < v_lim)
        wmask = lane0 & take

        sv = jnp.full((16,), s, jnp.int32)
        ev = jnp.full((16,), jnp.minimum(e, NT - 1), jnp.int32)
        plsc.store_scatter(sel_v, [jnp.full((16,), count, jnp.int32)],
                           jnp.full((16,), 9 * s + e, jnp.int32),
                           mask=wmask)
        new_me = jnp.maximum(me[0], e)
        plsc.store_scatter(s2e_v, [sv],
                           jnp.full((16,), new_me, jnp.int32),
                           mask=wmask)
        old_ms = plsc.load_gather(e2s_v, [ev])[0]
        new_ms = jnp.where((old_ms == -1) | (s < old_ms), s, old_ms)
        plsc.store_scatter(e2s_v, [ev],
                           jnp.full((16,), new_ms, jnp.int32),
                           mask=wmask)

        return count + jnp.where(take, 1, 0).astype(jnp.int32)

    return lax.fori_loop(i_lo, i_hi, body, count0)


def _is_main():
    return (lax.axis_index("c") == 0) & (lax.axis_index("s") == 0)


def _stage1_body(order_hbm, sel_hbm, cnt_hbm, s2e_hbm, e2s_hbm,
                 order_v, sel_v, cnt_v, s2e_v, e2s_v):
    pltpu.sync_copy(order_hbm.at[pl.ds(0, STAGE1)],
                    order_v.at[pl.ds(0, STAGE1)])
    neg1 = jnp.full((16,), -1, jnp.int32)

    def init_maps(j, carry):
        s2e_v[pl.ds(j * 16, 16)] = neg1
        e2s_v[pl.ds(j * 16, 16)] = neg1
        return carry

    lax.fori_loop(0, NT // 16, init_maps, 0)

    def init_sel(j, carry):
        sel_v[pl.ds(j * 16, 16)] = neg1
        return carry

    lax.fori_loop(0, SEL_PAD // 16, init_sel, 0)

    count = _greedy_loop(order_v, sel_v, s2e_v, e2s_v,
                         0, STAGE1, jnp.int32(0), None)
    cnt_v[pl.ds(0, 16)] = jnp.full((16,), count, jnp.int32)

    @pl.when(_is_main())
    def _():
        pltpu.sync_copy(sel_v, sel_hbm)
        pltpu.sync_copy(cnt_v, cnt_hbm)
        pltpu.sync_copy(s2e_v, s2e_hbm)
        pltpu.sync_copy(e2s_v, e2s_hbm)


def _stage2_body(order_hbm, vcnt_hbm, sel_in, cnt_in, s2e_in, e2s_in,
                 sel_hbm, cnt_hbm,
                 order_v, vcnt_v, sel_v, cnt_v, s2e_v, e2s_v):
    pltpu.sync_copy(order_hbm, order_v.at[pl.ds(0, NC)])
    pltpu.sync_copy(vcnt_hbm, vcnt_v)
    pltpu.sync_copy(sel_in, sel_v)
    pltpu.sync_copy(cnt_in, cnt_v)
    pltpu.sync_copy(s2e_in, s2e_v)
    pltpu.sync_copy(e2s_in, e2s_v)

    count0 = cnt_v[...][0]
    v_lim = vcnt_v[...][0]
    count = _greedy_loop(order_v, sel_v, s2e_v, e2s_v,
                         jnp.int32(STAGE1), jnp.int32(NC), count0, v_lim)
    cnt_v[pl.ds(0, 16)] = jnp.full((16,), count, jnp.int32)

    @pl.when(_is_main())
    def _():
        pltpu.sync_copy(sel_v, sel_hbm)
        pltpu.sync_copy(cnt_v, cnt_hbm)


_SC_PARAMS = dict(
    mesh=plsc.VectorSubcoreMesh(core_axis_name="c", subcore_axis_name="s"),
    compiler_params=pltpu.CompilerParams(needs_layout_passes=False),
)


def _stage1_call(order_packed):
    return pl.kernel(
        _stage1_body,
        out_type=(jax.ShapeDtypeStruct((SEL_PAD,), jnp.int32),
                  jax.ShapeDtypeStruct((16,), jnp.int32),
                  jax.ShapeDtypeStruct((NT,), jnp.int32),
                  jax.ShapeDtypeStruct((NT,), jnp.int32)),
        scratch_types=[
            pltpu.VMEM((STAGE1 + 16,), jnp.int32),
            pltpu.VMEM((SEL_PAD,), jnp.int32),
            pltpu.VMEM((16,), jnp.int32),
            pltpu.VMEM((NT,), jnp.int32),
            pltpu.VMEM((NT,), jnp.int32),
        ],
        **_SC_PARAMS,
    )(order_packed)


def _stage2_call(order_packed, vcnt, sel1, cnt1, s2e1, e2s1):
    return pl.kernel(
        _stage2_body,
        out_type=(jax.ShapeDtypeStruct((SEL_PAD,), jnp.int32),
                  jax.ShapeDtypeStruct((16,), jnp.int32)),
        scratch_types=[
            pltpu.VMEM((NC + 16,), jnp.int32),
            pltpu.VMEM((16,), jnp.int32),
            pltpu.VMEM((SEL_PAD,), jnp.int32),
            pltpu.VMEM((16,), jnp.int32),
            pltpu.VMEM((NT,), jnp.int32),
            pltpu.VMEM((NT,), jnp.int32),
        ],
        **_SC_PARAMS,
    )(order_packed, vcnt, sel1, cnt1, s2e1, e2s1)


def _emdot(a, b):
    return jnp.dot(a.astype(jnp.bfloat16), b.astype(jnp.bfloat16),
                   preferred_element_type=jnp.float32)


def kernel(token_emb, sentence_map, attn_w, attn_b, width_emb, W1, b1, W2, b2):
    N = token_emb.shape[0]
    t = jnp.arange(N, dtype=jnp.int32)
    dd = jnp.arange(MAXW, dtype=jnp.int32)
    starts = jnp.repeat(t, MAXW)
    ends = starts + jnp.tile(dd, N)
    start_sent = jnp.take(sentence_map, starts, axis=0)
    end_sent = jnp.take(sentence_map, jnp.minimum(ends, N - 1), axis=0)
    valid = (ends < N) & (start_sent == end_sent)
    ce_safe = jnp.minimum(ends, N - 1)

    start_emb = jnp.take(token_emb, starts, axis=0)
    end_emb = jnp.take(token_emb, ce_safe, axis=0)
    w_emb = jnp.take(width_emb, ce_safe - starts, axis=0)
    token_attn = _emdot(token_emb, attn_w) + attn_b
    doc_range = jnp.arange(N)[None, :]
    span_mask = (doc_range >= starts[:, None]) & (doc_range <= ce_safe[:, None])
    attn_logits = jnp.where(span_mask, token_attn[None, :], jnp.float32(-1e30))
    attn_probs = jax.nn.softmax(attn_logits, axis=1)
    attended = _emdot(attn_probs, token_emb)
    span_emb = jnp.concatenate([start_emb, end_emb, w_emb, attended], axis=1)
    h = jax.nn.relu(_emdot(span_emb, W1) + b1)
    scores = (_emdot(h, W2) + b2)[:, 0]

    num_top = int(0.4 * N)
    sort_key = jnp.where(valid, scores, jnp.float32(-jnp.inf))
    order = jnp.argsort(-sort_key)
    vcnt = jnp.full((16,), jnp.sum(valid.astype(jnp.int32)), jnp.int32)
    packed = ((jnp.take(starts, order) << 12) | jnp.take(ends, order)
              ).astype(jnp.int32)

    sel1, cnt1, s2e1, e2s1 = _stage1_call(packed)

    def _cont(_):
        return _stage2_call(packed, vcnt, sel1, cnt1, s2e1, e2s1)

    def _done(_):
        return sel1, cnt1

    sel_pad, cnt = lax.cond(cnt1[0] < NUM_TOP, _cont, _done, 0)
    sel = sel_pad[:num_top]
    count = cnt[0]

    slot = jnp.arange(num_top, dtype=jnp.int32)
    filled = slot < count
    key = jnp.where(filled,
                    jnp.take(starts, sel, mode='clip') * jnp.int32(N + MAXW)
                    + jnp.take(ends, sel, mode='clip'),
                    jnp.int32(jnp.iinfo(jnp.int32).max))
    perm = jnp.argsort(key)
    sel_sorted = jnp.take(sel, perm)
    sel_sorted = jnp.where(jnp.take(filled, perm), sel_sorted, sel_sorted[0])
    return (jnp.take(starts, sel_sorted), jnp.take(ends, sel_sorted),
            jnp.take(scores, sel_sorted))
